# baseline (speedup 1.0000x reference)
import functools

import jax
import jax.numpy as jnp
from jax import lax
from jax.experimental import pallas as pl
from jax.experimental.pallas import tpu as pltpu
from jax.experimental.pallas import tpu_sc as plsc

NSUB = 16
NCORE = 2
LANES = 16
CH = 1024
RB = 2048

_sc_mesh = functools.partial(
    plsc.VectorSubcoreMesh, core_axis_name="c", subcore_axis_name="s"
)
_SC_PARAMS = pltpu.CompilerParams(needs_layout_passes=False,
                                  use_tc_tiling_on_sc=False)


def _zero_vmem(ref, n):
    zeros = jnp.zeros((LANES,), jnp.float32)

    def body(i, _):
        ref[pl.ds(i * LANES, LANES)] = zeros
        return 0

    lax.fori_loop(0, n // LANES, body, 0, unroll=4)


def _deg_body(n_nodes, e_tile, col_hbm, ew_hbm, out_hbm, acc, colb, ewb):
    c = lax.axis_index("c")
    s = lax.axis_index("s")
    wid = s * NCORE + c
    _zero_vmem(acc, n_nodes)
    base = wid * e_tile

    def chunk(ci, _):
        off = base + ci * CH
        pltpu.sync_copy(col_hbm.at[pl.ds(off, CH)], colb)
        pltpu.sync_copy(ew_hbm.at[pl.ds(off, CH)], ewb)

        def grp(g, _):
            cv = colb[pl.ds(g * LANES, LANES)]
            wv = ewb[pl.ds(g * LANES, LANES)]
            plsc.addupdate_scatter(acc, [cv], wv)
            return 0

        lax.fori_loop(0, CH // LANES, grp, 0, unroll=4)
        return 0

    lax.fori_loop(0, e_tile // CH, chunk, 0)
    pltpu.sync_copy(acc, out_hbm.at[wid])


def _deg_partials(col_pad, ew_pad, n_nodes):
    e_pad = col_pad.shape[0]
    e_tile = e_pad // (NCORE * NSUB)
    return pl.kernel(
        functools.partial(_deg_body, n_nodes, e_tile),
        out_type=jax.ShapeDtypeStruct((NCORE * NSUB, n_nodes), jnp.float32),
        mesh=_sc_mesh(),
        compiler_params=_SC_PARAMS,
        scratch_types=[
            pltpu.VMEM((n_nodes,), jnp.float32),
            pltpu.VMEM((CH,), jnp.int32),
            pltpu.VMEM((CH,), jnp.float32),
        ],
    )(col_pad, ew_pad)


def _prop1_body(n_nodes, e_tile, t0_hbm, row_hbm, col_hbm, ew_hbm, out_hbm,
                xv, acc, rowb, colb, ewb):
    c = lax.axis_index("c")
    s = lax.axis_index("s")
    wid = s * NCORE + c
    pltpu.sync_copy(t0_hbm, xv)
    _zero_vmem(acc, n_nodes)
    base = wid * e_tile

    def chunk(ci, _):
        off = base + ci * CH
        pltpu.sync_copy(row_hbm.at[pl.ds(off, CH)], rowb)
        pltpu.sync_copy(col_hbm.at[pl.ds(off, CH)], colb)
        pltpu.sync_copy(ew_hbm.at[pl.ds(off, CH)], ewb)

        def grp(g, _):
            rv = rowb[pl.ds(g * LANES, LANES)]
            cv = colb[pl.ds(g * LANES, LANES)]
            wv = ewb[pl.ds(g * LANES, LANES)]
            xg = plsc.load_gather(xv, [rv])
            plsc.addupdate_scatter(acc, [cv], xg * wv)
            return 0

        lax.fori_loop(0, CH // LANES, grp, 0, unroll=4)
        return 0

    lax.fori_loop(0, e_tile // CH, chunk, 0)
    pltpu.sync_copy(acc, out_hbm.at[wid])


def _prop1(t0, row_pad, col_pad, ew_pad, n_nodes):
    e_pad = col_pad.shape[0]
    e_tile = e_pad // (NCORE * NSUB)
    return pl.kernel(
        functools.partial(_prop1_body, n_nodes, e_tile),
        out_type=jax.ShapeDtypeStruct((NCORE * NSUB, n_nodes), jnp.float32),
        mesh=_sc_mesh(),
        compiler_params=_SC_PARAMS,
        scratch_types=[
            pltpu.VMEM((n_nodes,), jnp.float32),
            pltpu.VMEM((n_nodes,), jnp.float32),
            pltpu.VMEM((CH,), jnp.int32),
            pltpu.VMEM((CH,), jnp.int32),
            pltpu.VMEM((CH,), jnp.float32),
        ],
    )(t0, row_pad, col_pad, ew_pad)


def _prop_body(n_nodes, dh, ch, e_tile, t_hbm, row_hbm, col_hbm, ew_hbm,
               out_hbm, acc, rowb0, colb0, gbuf0, rowb1, colb1,
               gbuf1, ewb, gsem0, gsem1, ssem0, ssem1):
    c = lax.axis_index("c")
    s = lax.axis_index("s")
    nrow = n_nodes // NSUB
    kd = ch // 128
    zeros = jnp.zeros((LANES,), jnp.float32)

    def zrow(i, _):
        for k in range(dh // LANES):
            gbuf0[i, pl.ds(k * LANES, LANES)] = zeros
        return 0

    lax.fori_loop(0, ch, zrow, 0, unroll=4)
    nfull = nrow // ch
    tail = nrow - nfull * ch
    for k in range(nfull):
        pltpu.sync_copy(gbuf0.at[pl.ds(0, ch)],
                        acc.at[pl.ds(s * nrow + k * ch, ch)])
    if tail:
        pltpu.sync_copy(gbuf0.at[pl.ds(0, tail)],
                        acc.at[pl.ds(s * nrow + nfull * ch, tail)])
    plsc.subcore_barrier()

    base = s * e_tile

    def load_idx(off, rb, cb):
        pltpu.sync_copy(row_hbm.at[c, pl.ds(off, ch)], rb)
        off128 = pl.multiple_of(off // 128, kd)
        pltpu.sync_copy(col_hbm.at[pl.ds(off128, kd)], cb)

    def do_mul(off, gb):
        pltpu.sync_copy(ew_hbm.at[pl.ds(off, ch)], ewb)

        def mul(g, _):
            wv = ewb[pl.ds(g * LANES, LANES)]
            for l in range(LANES):
                e = g * LANES + l
                w = wv[l]
                for k in range(dh // LANES):
                    v = gb[e, pl.ds(k * LANES, LANES)]
                    gb[e, pl.ds(k * LANES, LANES)] = v * w
            return 0

        lax.fori_loop(0, ch // LANES, mul, 0, unroll=4)

    def fire_scatter(gb, cb, sem):
        return [
            pltpu.async_copy(gb.at[pl.ds(j * 128, 128)],
                             acc.at[cb.at[j]], sem, add=True)
            for j in range(kd)
        ]

    def drain_scatter(sem):
        for _ in range(kd):
            pltpu.make_async_copy(t_hbm.at[pl.ds(0, 128)],
                                  acc.at[pl.ds(0, 128)], sem).wait()

    load_idx(base, rowb0, colb0)
    pltpu.async_copy(t_hbm.at[rowb0], gbuf0, gsem0)
    nit = e_tile // ch // 2

    def it_body(it, _):
        ci = base + it * (2 * ch)
        pltpu.make_async_copy(t_hbm.at[pl.ds(0, ch)], gbuf0, gsem0).wait()
        do_mul(ci, gbuf0)

        @pl.when(it > 0)
        def _():
            drain_scatter(ssem1)

        load_idx(ci + ch, rowb1, colb1)
        g1 = pltpu.async_copy(t_hbm.at[rowb1], gbuf1, gsem1)
        s0 = fire_scatter(gbuf0, colb0, ssem0)
        g1.wait()
        do_mul(ci + ch, gbuf1)
        for cp in s0:
            cp.wait()

        @pl.when(it < nit - 1)
        def _():
            load_idx(ci + 2 * ch, rowb0, colb0)
            pltpu.async_copy(t_hbm.at[rowb0], gbuf0, gsem0)

        fire_scatter(gbuf1, colb1, ssem1)
        return 0

    lax.fori_loop(0, nit, it_body, 0)
    drain_scatter(ssem1)
    plsc.subcore_barrier()
    pltpu.sync_copy(acc.at[pl.ds(s * nrow, nrow)],
                    out_hbm.at[c, pl.ds(s * nrow, nrow)])


def _prop(t2, row2_pad, col2d_pad, ew_pad, n_nodes):
    dh = t2.shape[2]
    e_pad = ew_pad.shape[0]
    e_tile = e_pad // NSUB
    t_flat = t2.reshape(NCORE * n_nodes, dh)
    ch = CH
    buf_set = [
        pltpu.VMEM((ch,), jnp.int32),
        pltpu.VMEM((ch // 128, 128), jnp.int32),
        pltpu.VMEM((ch, dh), jnp.float32),
    ]
    return pl.kernel(
        functools.partial(_prop_body, n_nodes, dh, ch, e_tile),
        out_type=jax.ShapeDtypeStruct((NCORE, n_nodes, dh), jnp.float32),
        mesh=_sc_mesh(),
        compiler_params=_SC_PARAMS,
        scratch_types=[pltpu.VMEM_SHARED((n_nodes, dh), jnp.float32)]
        + buf_set + buf_set
        + [pltpu.VMEM((ch,), jnp.float32)]
        + [pltpu.SemaphoreType.DMA] * 4,
    )(t_flat, row2_pad, col2d_pad, ew_pad)


def _col_sum(p_blk):
    ones = jnp.ones((p_blk.shape[0], 1), jnp.float32)
    return lax.dot_general(p_blk, ones, (((0,), (0,)), ((), ())),
                           preferred_element_type=jnp.float32)


def _tc_call(body, n_nodes, out_shapes, in_arrays, in_specs, out_specs):
    return pl.pallas_call(
        body,
        grid=(n_nodes // RB,),
        out_shape=out_shapes,
        in_specs=in_specs,
        out_specs=out_specs,
    )(*in_arrays)


def _spec_rows(*dims):
    blk = (RB,) + dims
    return pl.BlockSpec(blk, lambda i: (i,) + (0,) * len(dims))


def _spec_halves(dh):
    return pl.BlockSpec((NCORE, RB, dh), lambda i: (0, i, 0))


def _spec_quarters():
    return pl.BlockSpec((4, RB, 16), lambda i: (0, i, 0))


def _spec_part(k):
    return pl.BlockSpec((k, RB), lambda i: (0, i))


def _spec_full(a):
    return pl.BlockSpec(a.shape, lambda i: (0,) * a.ndim)


def _split_halves(out_ref, t):
    dh = t.shape[1] // 2
    out_ref[0] = t[:, :dh]
    out_ref[1] = t[:, dh:]


def _split_quarters(out_ref, t):
    for i in range(4):
        out_ref[i] = t[:, i * 16:(i + 1) * 16]


def _tcB_body(part_ref, x_ref, dis_ref, t0_ref):
    deg = _col_sum(part_ref[...])
    dis = jnp.where(deg > 0, lax.rsqrt(deg), 0.0)
    dis_ref[...] = dis
    t0_ref[...] = x_ref[...] * dis


def _tcD_body(part_ref, dis_ref, w1_ref, b1_ref, out_ref):
    dis = dis_ref[...]
    p0 = dis * _col_sum(part_ref[...])
    h1 = jax.nn.sigmoid(
        jnp.dot(p0, w1_ref[...], preferred_element_type=jnp.float32)
        + b1_ref[...])
    _split_halves(out_ref, dis * h1)


def _tcF_body(q_ref, dis_ref, w2_ref, b2_ref, out_ref):
    dis = dis_ref[...]
    q = jnp.concatenate([q_ref[0], q_ref[1]], axis=1)
    h2 = jax.nn.relu(
        jnp.dot(dis * q, w2_ref[...], preferred_element_type=jnp.float32)
        + b2_ref[...])
    _split_quarters(out_ref, dis * h2)


def _tcH_body(qa_ref, qb_ref, dis_ref, w3_ref, b3_ref, w4_ref, out_ref):
    dis = dis_ref[...]
    q = jnp.concatenate([qa_ref[0], qa_ref[1], qb_ref[0], qb_ref[1]], axis=1)
    h3 = jax.nn.sigmoid(
        jnp.dot(dis * q, w3_ref[...], preferred_element_type=jnp.float32)
        + b3_ref[...])
    g4 = jnp.dot(h3, w4_ref[...], preferred_element_type=jnp.float32)
    _split_quarters(out_ref, dis * g4)


def _tcJ_body(qa_ref, qb_ref, dis_ref, b4_ref, w5_ref, out_ref):
    dis = dis_ref[...]
    q = jnp.concatenate([qa_ref[0], qa_ref[1], qb_ref[0], qb_ref[1]], axis=1)
    h4 = jax.nn.relu(dis * q + b4_ref[...])
    g5 = jnp.dot(h4, w5_ref[...], preferred_element_type=jnp.float32)
    _split_halves(out_ref, dis * g5)


def _tcL_body(q_ref, dis_ref, b5_ref, out_ref):
    dis = dis_ref[...]
    q = jnp.concatenate([q_ref[0], q_ref[1]], axis=1)
    out_ref[...] = jax.nn.sigmoid(dis * q + b5_ref[...])


def kernel(x, edge_index, edge_weights, W1, b1, W2, b2, W3, b3, W4, b4, W5, b5):
    n_real = x.shape[0]
    n_nodes = ((n_real + RB - 1) // RB) * RB
    n_edges = edge_weights.shape[0]
    row = edge_index[0]
    col = edge_index[1]
    x = jnp.pad(x, ((0, n_nodes - n_real), (0, 0)))

    e_pad = ((n_edges + NCORE * NSUB * CH - 1) // (NCORE * NSUB * CH)) * (
        NCORE * NSUB * CH)
    padn = e_pad - n_edges
    row_p = jnp.pad(row, (0, padn))
    col_p = jnp.pad(col, (0, padn))
    ew_p = jnp.pad(edge_weights, (0, padn))
    row2 = jnp.stack([row_p, row_p + n_nodes])
    col2d = col_p.reshape(e_pad // 128, 128)

    b1r, b2r, b3r, b4r, b5r = (b.reshape(1, -1) for b in (b1, b2, b3, b4, b5))

    deg_part = _deg_partials(col_p, ew_p, n_nodes)

    dis, t0 = _tc_call(
        _tcB_body, n_nodes,
        (jax.ShapeDtypeStruct((n_nodes, 1), jnp.float32),
         jax.ShapeDtypeStruct((n_nodes, 1), jnp.float32)),
        (deg_part, x),
        [_spec_part(NCORE * NSUB), _spec_rows(1)],
        (_spec_rows(1), _spec_rows(1)),
    )

    q0 = _prop1(t0.reshape(n_nodes), row_p, col_p, ew_p, n_nodes)

    t1 = _tc_call(
        _tcD_body, n_nodes,
        jax.ShapeDtypeStruct((NCORE, n_nodes, 16), jnp.float32),
        (q0, dis, W1, b1r),
        [_spec_part(NCORE * NSUB), _spec_rows(1), _spec_full(W1),
         _spec_full(b1r)],
        _spec_halves(16),
    )

    q1 = _prop(t1, row2, col2d, ew_p, n_nodes)

    t2 = _tc_call(
        _tcF_body, n_nodes,
        jax.ShapeDtypeStruct((4, n_nodes, 16), jnp.float32),
        (q1, dis, W2, b2r),
        [_spec_halves(16), _spec_rows(1), _spec_full(W2), _spec_full(b2r)],
        _spec_quarters(),
    )

    q2a = _prop(t2[:2], row2, col2d, ew_p, n_nodes)
    q2b = _prop(t2[2:], row2, col2d, ew_p, n_nodes)

    t3 = _tc_call(
        _tcH_body, n_nodes,
        jax.ShapeDtypeStruct((4, n_nodes, 16), jnp.float32),
        (q2a, q2b, dis, W3, b3r, W4),
        [_spec_halves(16), _spec_halves(16), _spec_rows(1), _spec_full(W3),
         _spec_full(b3r), _spec_full(W4)],
        _spec_quarters(),
    )

    q3a = _prop(t3[:2], row2, col2d, ew_p, n_nodes)
    q3b = _prop(t3[2:], row2, col2d, ew_p, n_nodes)

    t4 = _tc_call(
        _tcJ_body, n_nodes,
        jax.ShapeDtypeStruct((NCORE, n_nodes, 16), jnp.float32),
        (q3a, q3b, dis, b4r, W5),
        [_spec_halves(16), _spec_halves(16), _spec_rows(1), _spec_full(b4r),
         _spec_full(W5)],
        _spec_halves(16),
    )

    q4 = _prop(t4, row2, col2d, ew_p, n_nodes)

    out = _tc_call(
        _tcL_body, n_nodes,
        jax.ShapeDtypeStruct((n_nodes, 32), jnp.float32),
        (q4, dis, b5r),
        [_spec_halves(16), _spec_rows(1), _spec_full(b5r)],
        _spec_rows(32),
    )
    return out[:n_real]


# speedup vs baseline: 1.3013x; 1.3013x over previous
import functools

import jax
import jax.numpy as jnp
from jax import lax
from jax.experimental import pallas as pl
from jax.experimental.pallas import tpu as pltpu
from jax.experimental.pallas import tpu_sc as plsc

NSUB = 16
NCORE = 2
LANES = 16
CH = 1024
RB = 2048

_sc_mesh = functools.partial(
    plsc.VectorSubcoreMesh, core_axis_name="c", subcore_axis_name="s"
)
_SC_PARAMS = pltpu.CompilerParams(needs_layout_passes=False,
                                  use_tc_tiling_on_sc=False)


def _zero_vmem(ref, n):
    zeros = jnp.zeros((LANES,), jnp.float32)

    def body(i, _):
        ref[pl.ds(i * LANES, LANES)] = zeros
        return 0

    lax.fori_loop(0, n // LANES, body, 0, unroll=4)


def _deg_body(n_nodes, e_tile, col_hbm, ew_hbm, out_hbm, acc, colb, ewb):
    c = lax.axis_index("c")
    s = lax.axis_index("s")
    wid = s * NCORE + c
    _zero_vmem(acc, n_nodes)
    base = wid * e_tile

    def chunk(ci, _):
        off = base + ci * CH
        pltpu.sync_copy(col_hbm.at[pl.ds(off, CH)], colb)
        pltpu.sync_copy(ew_hbm.at[pl.ds(off, CH)], ewb)

        def grp(g, _):
            cv = colb[pl.ds(g * LANES, LANES)]
            wv = ewb[pl.ds(g * LANES, LANES)]
            plsc.addupdate_scatter(acc, [cv], wv)
            return 0

        lax.fori_loop(0, CH // LANES, grp, 0, unroll=4)
        return 0

    lax.fori_loop(0, e_tile // CH, chunk, 0)
    pltpu.sync_copy(acc, out_hbm.at[wid])


def _deg_partials(col_pad, ew_pad, n_nodes):
    e_pad = col_pad.shape[0]
    e_tile = e_pad // (NCORE * NSUB)
    return pl.kernel(
        functools.partial(_deg_body, n_nodes, e_tile),
        out_type=jax.ShapeDtypeStruct((NCORE * NSUB, n_nodes), jnp.float32),
        mesh=_sc_mesh(),
        compiler_params=_SC_PARAMS,
        scratch_types=[
            pltpu.VMEM((n_nodes,), jnp.float32),
            pltpu.VMEM((CH,), jnp.int32),
            pltpu.VMEM((CH,), jnp.float32),
        ],
    )(col_pad, ew_pad)


def _prop1_body(n_nodes, e_tile, t0_hbm, row_hbm, col_hbm, ew_hbm, out_hbm,
                xv, acc, rowb, colb, ewb):
    c = lax.axis_index("c")
    s = lax.axis_index("s")
    wid = s * NCORE + c
    pltpu.sync_copy(t0_hbm, xv)
    _zero_vmem(acc, n_nodes)
    base = wid * e_tile

    def chunk(ci, _):
        off = base + ci * CH
        pltpu.sync_copy(row_hbm.at[pl.ds(off, CH)], rowb)
        pltpu.sync_copy(col_hbm.at[pl.ds(off, CH)], colb)
        pltpu.sync_copy(ew_hbm.at[pl.ds(off, CH)], ewb)

        def grp(g, _):
            rv = rowb[pl.ds(g * LANES, LANES)]
            cv = colb[pl.ds(g * LANES, LANES)]
            wv = ewb[pl.ds(g * LANES, LANES)]
            xg = plsc.load_gather(xv, [rv])
            plsc.addupdate_scatter(acc, [cv], xg * wv)
            return 0

        lax.fori_loop(0, CH // LANES, grp, 0, unroll=4)
        return 0

    lax.fori_loop(0, e_tile // CH, chunk, 0)
    pltpu.sync_copy(acc, out_hbm.at[wid])


def _prop1(t0, row_pad, col_pad, ew_pad, n_nodes):
    e_pad = col_pad.shape[0]
    e_tile = e_pad // (NCORE * NSUB)
    return pl.kernel(
        functools.partial(_prop1_body, n_nodes, e_tile),
        out_type=jax.ShapeDtypeStruct((NCORE * NSUB, n_nodes), jnp.float32),
        mesh=_sc_mesh(),
        compiler_params=_SC_PARAMS,
        scratch_types=[
            pltpu.VMEM((n_nodes,), jnp.float32),
            pltpu.VMEM((n_nodes,), jnp.float32),
            pltpu.VMEM((CH,), jnp.int32),
            pltpu.VMEM((CH,), jnp.int32),
            pltpu.VMEM((CH,), jnp.float32),
        ],
    )(t0, row_pad, col_pad, ew_pad)


def _prop_body(n_nodes, dh, ch, e_tile, t_hbm, row_hbm, col_hbm, ew_hbm,
               out_hbm, acc, rowb0, colb0, gbuf0, rowb1, colb1,
               gbuf1, ewb, gsem0, gsem1, ssem0, ssem1):
    c = lax.axis_index("c")
    s = lax.axis_index("s")
    nrow = n_nodes // NSUB
    kd = ch // 128
    zeros = jnp.zeros((LANES,), jnp.float32)

    def zrow(i, _):
        for k in range(dh // LANES):
            gbuf0[i, pl.ds(k * LANES, LANES)] = zeros
        return 0

    lax.fori_loop(0, ch, zrow, 0, unroll=4)
    nfull = nrow // ch
    tail = nrow - nfull * ch
    for k in range(nfull):
        pltpu.sync_copy(gbuf0.at[pl.ds(0, ch)],
                        acc.at[pl.ds(s * nrow + k * ch, ch)])
    if tail:
        pltpu.sync_copy(gbuf0.at[pl.ds(0, tail)],
                        acc.at[pl.ds(s * nrow + nfull * ch, tail)])
    plsc.subcore_barrier()

    base = s * e_tile

    def load_idx(off, rb, cb):
        pltpu.sync_copy(row_hbm.at[c, pl.ds(off, ch)], rb)
        off128 = pl.multiple_of(off // 128, kd)
        pltpu.sync_copy(col_hbm.at[pl.ds(off128, kd)], cb)

    def do_mul(off, gb):
        pltpu.sync_copy(ew_hbm.at[pl.ds(off, ch)], ewb)

        def mul(g, _):
            wv = ewb[pl.ds(g * LANES, LANES)]
            for l in range(LANES):
                e = g * LANES + l
                w = wv[l]
                for k in range(dh // LANES):
                    v = gb[e, pl.ds(k * LANES, LANES)]
                    gb[e, pl.ds(k * LANES, LANES)] = v * w
            return 0

        lax.fori_loop(0, ch // LANES, mul, 0, unroll=2)

    def fire_scatter(gb, cb, sem):
        return [
            pltpu.async_copy(gb.at[pl.ds(j * 128, 128)],
                             acc.at[cb.at[j]], sem, add=True)
            for j in range(kd)
        ]

    def drain_scatter(sem):
        for _ in range(kd):
            pltpu.make_async_copy(t_hbm.at[pl.ds(0, 128)],
                                  acc.at[pl.ds(0, 128)], sem).wait()

    load_idx(base, rowb0, colb0)
    pltpu.async_copy(t_hbm.at[rowb0], gbuf0, gsem0)
    nit = e_tile // ch // 2

    def it_body(it, _):
        ci = base + it * (2 * ch)
        pltpu.make_async_copy(t_hbm.at[pl.ds(0, ch)], gbuf0, gsem0).wait()
        do_mul(ci, gbuf0)

        @pl.when(it > 0)
        def _():
            drain_scatter(ssem1)

        load_idx(ci + ch, rowb1, colb1)
        g1 = pltpu.async_copy(t_hbm.at[rowb1], gbuf1, gsem1)
        s0 = fire_scatter(gbuf0, colb0, ssem0)
        g1.wait()
        do_mul(ci + ch, gbuf1)
        for cp in s0:
            cp.wait()

        @pl.when(it < nit - 1)
        def _():
            load_idx(ci + 2 * ch, rowb0, colb0)
            pltpu.async_copy(t_hbm.at[rowb0], gbuf0, gsem0)

        fire_scatter(gbuf1, colb1, ssem1)
        return 0

    lax.fori_loop(0, nit, it_body, 0)
    drain_scatter(ssem1)
    plsc.subcore_barrier()
    pltpu.sync_copy(acc.at[pl.ds(s * nrow, nrow)],
                    out_hbm.at[c, pl.ds(s * nrow, nrow)])


def _prop(t2, row2_pad, col2d_pad, ew_pad, n_nodes):
    dh = t2.shape[2]
    e_pad = ew_pad.shape[0]
    e_tile = e_pad // NSUB
    t_flat = t2.reshape(NCORE * n_nodes, dh)
    ch = CH
    buf_set = [
        pltpu.VMEM((ch,), jnp.int32),
        pltpu.VMEM((ch // 128, 128), jnp.int32),
        pltpu.VMEM((ch, dh), jnp.float32),
    ]
    return pl.kernel(
        functools.partial(_prop_body, n_nodes, dh, ch, e_tile),
        out_type=jax.ShapeDtypeStruct((NCORE, n_nodes, dh), jnp.float32),
        mesh=_sc_mesh(),
        compiler_params=_SC_PARAMS,
        scratch_types=[pltpu.VMEM_SHARED((n_nodes, dh), jnp.float32)]
        + buf_set + buf_set
        + [pltpu.VMEM((ch,), jnp.float32)]
        + [pltpu.SemaphoreType.DMA] * 4,
    )(t_flat, row2_pad, col2d_pad, ew_pad)


def _col_sum(p_blk):
    ones = jnp.ones((p_blk.shape[0], 1), jnp.float32)
    return lax.dot_general(p_blk, ones, (((0,), (0,)), ((), ())),
                           preferred_element_type=jnp.float32)


def _tc_call(body, n_nodes, out_shapes, in_arrays, in_specs, out_specs):
    return pl.pallas_call(
        body,
        grid=(n_nodes // RB,),
        out_shape=out_shapes,
        in_specs=in_specs,
        out_specs=out_specs,
    )(*in_arrays)


def _spec_rows(*dims):
    blk = (RB,) + dims
    return pl.BlockSpec(blk, lambda i: (i,) + (0,) * len(dims))


def _spec_halves(dh):
    return pl.BlockSpec((NCORE, RB, dh), lambda i: (0, i, 0))


def _spec_quarters():
    return pl.BlockSpec((4, RB, 16), lambda i: (0, i, 0))


def _spec_part(k):
    return pl.BlockSpec((k, RB), lambda i: (0, i))


def _spec_full(a):
    return pl.BlockSpec(a.shape, lambda i: (0,) * a.ndim)


def _split_halves(out_ref, t):
    dh = t.shape[1] // 2
    out_ref[0] = t[:, :dh]
    out_ref[1] = t[:, dh:]


def _split_quarters(out_ref, t):
    for i in range(4):
        out_ref[i] = t[:, i * 16:(i + 1) * 16]


def _tcB_body(part_ref, x_ref, dis_ref, t0_ref):
    deg = _col_sum(part_ref[...])
    dis = jnp.where(deg > 0, lax.rsqrt(deg), 0.0)
    dis_ref[...] = dis
    t0_ref[...] = x_ref[...] * dis


def _tcD_body(part_ref, dis_ref, w1_ref, b1_ref, out_ref):
    dis = dis_ref[...]
    p0 = dis * _col_sum(part_ref[...])
    h1 = jax.nn.sigmoid(
        jnp.dot(p0, w1_ref[...], preferred_element_type=jnp.float32)
        + b1_ref[...])
    _split_halves(out_ref, dis * h1)


def _tcF_body(q_ref, dis_ref, w2_ref, b2_ref, out_ref):
    dis = dis_ref[...]
    q = jnp.concatenate([q_ref[0], q_ref[1]], axis=1)
    h2 = jax.nn.relu(
        jnp.dot(dis * q, w2_ref[...], preferred_element_type=jnp.float32)
        + b2_ref[...])
    _split_quarters(out_ref, dis * h2)


def _tcH_body(qa_ref, qb_ref, dis_ref, w3_ref, b3_ref, w4_ref, out_ref):
    dis = dis_ref[...]
    q = jnp.concatenate([qa_ref[0], qa_ref[1], qb_ref[0], qb_ref[1]], axis=1)
    h3 = jax.nn.sigmoid(
        jnp.dot(dis * q, w3_ref[...], preferred_element_type=jnp.float32)
        + b3_ref[...])
    g4 = jnp.dot(h3, w4_ref[...], preferred_element_type=jnp.float32)
    _split_quarters(out_ref, dis * g4)


def _tcJ_body(qa_ref, qb_ref, dis_ref, b4_ref, w5_ref, out_ref):
    dis = dis_ref[...]
    q = jnp.concatenate([qa_ref[0], qa_ref[1], qb_ref[0], qb_ref[1]], axis=1)
    h4 = jax.nn.relu(dis * q + b4_ref[...])
    g5 = jnp.dot(h4, w5_ref[...], preferred_element_type=jnp.float32)
    _split_halves(out_ref, dis * g5)


def _tcL_body(q_ref, dis_ref, b5_ref, out_ref):
    dis = dis_ref[...]
    q = jnp.concatenate([q_ref[0], q_ref[1]], axis=1)
    out_ref[...] = jax.nn.sigmoid(dis * q + b5_ref[...])


def kernel(x, edge_index, edge_weights, W1, b1, W2, b2, W3, b3, W4, b4, W5, b5):
    n_real = x.shape[0]
    n_nodes = ((n_real + RB - 1) // RB) * RB
    n_edges = edge_weights.shape[0]
    row = edge_index[0]
    col = edge_index[1]
    x = jnp.pad(x, ((0, n_nodes - n_real), (0, 0)))

    e_pad = ((n_edges + NCORE * NSUB * CH - 1) // (NCORE * NSUB * CH)) * (
        NCORE * NSUB * CH)
    padn = e_pad - n_edges
    row_p = jnp.pad(row, (0, padn))
    col_p = jnp.pad(col, (0, padn))
    ew_p = jnp.pad(edge_weights, (0, padn))
    row2 = jnp.stack([row_p, row_p + n_nodes])
    col2d = col_p.reshape(e_pad // 128, 128)

    b1r, b2r, b3r, b4r, b5r = (b.reshape(1, -1) for b in (b1, b2, b3, b4, b5))

    deg_part = _deg_partials(col_p, ew_p, n_nodes)

    dis, t0 = _tc_call(
        _tcB_body, n_nodes,
        (jax.ShapeDtypeStruct((n_nodes, 1), jnp.float32),
         jax.ShapeDtypeStruct((n_nodes, 1), jnp.float32)),
        (deg_part, x),
        [_spec_part(NCORE * NSUB), _spec_rows(1)],
        (_spec_rows(1), _spec_rows(1)),
    )

    q0 = _prop1(t0.reshape(n_nodes), row_p, col_p, ew_p, n_nodes)

    t1 = _tc_call(
        _tcD_body, n_nodes,
        jax.ShapeDtypeStruct((NCORE, n_nodes, 16), jnp.float32),
        (q0, dis, W1, b1r),
        [_spec_part(NCORE * NSUB), _spec_rows(1), _spec_full(W1),
         _spec_full(b1r)],
        _spec_halves(16),
    )

    q1 = _prop(t1, row2, col2d, ew_p, n_nodes)

    t2 = _tc_call(
        _tcF_body, n_nodes,
        jax.ShapeDtypeStruct((4, n_nodes, 16), jnp.float32),
        (q1, dis, W2, b2r),
        [_spec_halves(16), _spec_rows(1), _spec_full(W2), _spec_full(b2r)],
        _spec_quarters(),
    )

    q2a = _prop(t2[:2], row2, col2d, ew_p, n_nodes)
    q2b = _prop(t2[2:], row2, col2d, ew_p, n_nodes)

    t3 = _tc_call(
        _tcH_body, n_nodes,
        jax.ShapeDtypeStruct((4, n_nodes, 16), jnp.float32),
        (q2a, q2b, dis, W3, b3r, W4),
        [_spec_halves(16), _spec_halves(16), _spec_rows(1), _spec_full(W3),
         _spec_full(b3r), _spec_full(W4)],
        _spec_quarters(),
    )

    q3a = _prop(t3[:2], row2, col2d, ew_p, n_nodes)
    q3b = _prop(t3[2:], row2, col2d, ew_p, n_nodes)

    t4 = _tc_call(
        _tcJ_body, n_nodes,
        jax.ShapeDtypeStruct((NCORE, n_nodes, 16), jnp.float32),
        (q3a, q3b, dis, b4r, W5),
        [_spec_halves(16), _spec_halves(16), _spec_rows(1), _spec_full(b4r),
         _spec_full(W5)],
        _spec_halves(16),
    )

    q4 = _prop(t4, row2, col2d, ew_p, n_nodes)

    out = _tc_call(
        _tcL_body, n_nodes,
        jax.ShapeDtypeStruct((n_nodes, 32), jnp.float32),
        (q4, dis, b5r),
        [_spec_halves(16), _spec_rows(1), _spec_full(b5r)],
        _spec_rows(32),
    )
    return out[:n_real]
